# baseline (speedup 1.0000x reference)
import functools

import jax
import jax.numpy as jnp
from jax import lax
from jax.experimental import pallas as pl
from jax.experimental.pallas import tpu as pltpu
from jax.experimental.pallas import tpu_sc as plsc

B, T, K = 16, 512, 64


def _tc_logz_body(emis_ref, trans_ref, transT_ref, start_ref, end_ref,
                  len_ref, out_ref, E_ref):
    expA_bf = jnp.exp(trans_ref[...]).astype(jnp.bfloat16)
    expAT_bf = jnp.exp(transT_ref[...]).astype(jnp.bfloat16)
    lens = len_ref[...]

    H = T // 2
    lens3 = lens[:, :, None]
    emsum_f = jnp.zeros((B, 1), jnp.float32)
    emsum_b = jnp.zeros((B, 1), jnp.float32)
    CH = 128
    for ci in range(T // CH):
        e = emis_ref[:, pl.ds(ci * CH, CH), :]
        em = jnp.max(e, axis=2, keepdims=True)
        E_ref[:, pl.ds(ci * CH, CH), :] = jnp.exp(e - em)
        ti = lax.broadcasted_iota(jnp.int32, (B, CH, 1), 1) + ci * CH
        mk = (ti >= 1) & (ti < lens3)
        emv = jnp.where(mk, em, 0.0)
        if ci < (T // CH) // 2:
            emsum_f = emsum_f + jnp.sum(emv, axis=1)
        else:
            emsum_b = emsum_b + jnp.sum(emv, axis=1)

    alpha0 = start_ref[...] + emis_ref[:, 0, :]
    m0 = jnp.max(alpha0, axis=1, keepdims=True)
    a = jnp.exp(alpha0 - m0)
    ca = m0 + emsum_f

    endv = end_ref[...]
    mb0 = jnp.max(endv)
    b = jnp.broadcast_to(jnp.exp(endv - mb0), (B, K))
    cb = mb0 + emsum_b

    def fsub(t, a):
        E = E_ref[:, t, :]
        an = jnp.dot(a.astype(jnp.bfloat16), expA_bf,
                     preferred_element_type=jnp.float32) * E
        return jnp.where(t < lens, an, a)

    def bsub(t, b):
        E = E_ref[:, t, :]
        bn = jnp.dot((b * E).astype(jnp.bfloat16), expAT_bf,
                     preferred_element_type=jnp.float32)
        return jnp.where(t < lens, bn, b)

    def renorm(x, c):
        m = jnp.max(x, axis=1, keepdims=True)
        return x * (1.0 / m), c + jnp.log(m)

    for j in range(1, 8):
        a = fsub(j, a)
    for j in range(4):
        b = bsub(T - 1 - j, b)
    b, cb = renorm(b, cb)
    for j in range(4, 8):
        b = bsub(T - 1 - j, b)
    a, ca = renorm(a, ca)
    b, cb = renorm(b, cb)

    def group(g, carry):
        a, ca, b, cb = carry
        for j in range(4):
            a = fsub(8 * g + j, a)
            b = bsub(T - 1 - 8 * g - j, b)
        b, cb = renorm(b, cb)
        for j in range(4, 8):
            a = fsub(8 * g + j, a)
            b = bsub(T - 1 - 8 * g - j, b)
        a, ca = renorm(a, ca)
        b, cb = renorm(b, cb)
        return a, ca, b, cb

    a, ca, b, cb = lax.fori_loop(1, H // 8, group, (a, ca, b, cb))
    z = ca[:, 0] + cb[:, 0] + jnp.log(jnp.sum(a * b, axis=1))
    out_ref[...] = jnp.broadcast_to(z[:, None], (B, 128))


def _tc_logz(emissions, transitions, start_transitions, end_transitions,
             len32):
    out = pl.pallas_call(
        _tc_logz_body,
        out_shape=jax.ShapeDtypeStruct((B, 128), jnp.float32),
        scratch_shapes=[pltpu.VMEM((B, T, K), jnp.float32)],
    )(
        emissions,
        transitions,
        transitions.T,
        start_transitions.reshape(1, K),
        end_transitions.reshape(1, K),
        len32.reshape(B, 1),
    )
    return out[:, 0]


@functools.cache
def _make_sc_scores():
    mesh = plsc.VectorSubcoreMesh(core_axis_name="c", subcore_axis_name="s")

    HT = T // 2

    @functools.partial(
        pl.kernel,
        mesh=mesh,
        compiler_params=pltpu.CompilerParams(needs_layout_passes=False),
        out_type=jax.ShapeDtypeStruct((2 * B, 16), jnp.float32),
        scratch_types=[
            pltpu.VMEM((HT, K), jnp.float32),
            pltpu.VMEM((T,), jnp.int32),
            pltpu.VMEM((K, K), jnp.float32),
            pltpu.VMEM((K,), jnp.float32),
            pltpu.VMEM((K,), jnp.float32),
            pltpu.VMEM((B,), jnp.int32),
            pltpu.VMEM((16,), jnp.float32),
            pltpu.SemaphoreType.DMA,
        ],
    )
    def _sc_scores(emis_hbm, trans_hbm, start_hbm, end_hbm, tags_hbm, len_hbm,
                   out_hbm, emis_v, tags_v, trans_v, start_v, end_v, len_v,
                   acc_v, sem):
        s_id = lax.axis_index("s")
        c_id = lax.axis_index("c")
        wid = s_id * 2 + c_id
        b = s_id
        h = c_id

        cps = [
            pltpu.async_copy(emis_hbm.at[b, pl.ds(h * HT, HT)], emis_v, sem),
            pltpu.async_copy(tags_hbm.at[b], tags_v, sem),
            pltpu.async_copy(trans_hbm, trans_v, sem),
            pltpu.async_copy(start_hbm, start_v, sem),
            pltpu.async_copy(end_hbm, end_v, sem),
            pltpu.async_copy(len_hbm, len_v, sem),
        ]
        for cp in cps:
            cp.wait()

        lane = lax.iota(jnp.int32, 16)
        bvec = jnp.full((16,), b, jnp.int32)
        lenv = plsc.load_gather(len_v, [bvec])
        hoff = h * HT

        def gath(i, acc):
            tl_ = lane + 16 * i
            tv = tl_ + hoff
            tags_t = plsc.load_gather(tags_v, [tv])
            ev = plsc.load_gather(emis_v, [tl_, tags_t])
            mk = (tv < lenv).astype(jnp.float32)
            acc = acc + ev * mk
            pv = plsc.load_gather(tags_v, [jnp.maximum(tv - 1, 0)])
            trv = plsc.load_gather(trans_v, [pv, tags_t])
            mk2 = jnp.where(tv >= 1, mk, 0.0)
            return acc + trv * mk2

        acc = lax.fori_loop(0, HT // 16, gath, jnp.zeros((16,), jnp.float32))

        lane0 = lane == 0
        zero16 = jnp.zeros((16,), jnp.int32)
        t0 = plsc.load_gather(tags_v, [zero16])
        sv = plsc.load_gather(start_v, [t0])
        smk = (lane0 & (jnp.full((16,), h) == 0)).astype(jnp.float32)
        acc = acc + sv * smk
        lastt = jnp.maximum(lenv - 1, 0)
        tlast = plsc.load_gather(tags_v, [lastt])
        evv = plsc.load_gather(end_v, [tlast])
        emk = (lane0 & (lastt // HT == jnp.full((16,), h))).astype(jnp.float32)
        acc = acc + evv * emk

        tot = jnp.sum(acc)
        acc_v[...] = jnp.full((16,), tot)
        pltpu.sync_copy(acc_v, out_hbm.at[wid])

    return _sc_scores


def kernel(emissions, transitions, start_transitions, end_transitions, tags,
           lengths):
    tags32 = tags.astype(jnp.int32)
    len32 = lengths.astype(jnp.int32)
    log_z = _tc_logz(emissions, transitions, start_transitions, end_transitions,
                     len32)
    sc_out = _make_sc_scores()(emissions, transitions, start_transitions,
                               end_transitions, tags32, len32)
    scores = sc_out[:, 0].reshape(B, 2).sum(axis=1)
    return scores - log_z


# speedup vs baseline: 1.1836x; 1.1836x over previous
import functools

import jax
import jax.numpy as jnp
from jax import lax
from jax.experimental import pallas as pl
from jax.experimental.pallas import tpu as pltpu
from jax.experimental.pallas import tpu_sc as plsc

B, T, K = 16, 512, 64


def _tc_logz_body(emis_ref, trans_ref, transT_ref, start_ref, end_ref,
                  len_ref, out_ref, E_ref, Ew_ref):
    expA_bf = jnp.exp(trans_ref[...]).astype(jnp.bfloat16)
    expAT_bf = jnp.exp(transT_ref[...]).astype(jnp.bfloat16)
    lens = len_ref[...]

    H = T // 2
    lens3 = lens[:, :, None]
    emsum_f = jnp.zeros((B, 1), jnp.float32)
    emsum_b = jnp.zeros((B, 1), jnp.float32)
    CH = 128
    for ci in range(T // CH):
        e = emis_ref[:, pl.ds(ci * CH, CH), :]
        em = jnp.max(e, axis=2, keepdims=True)
        Ech = jnp.exp(e - em)
        E_ref[:, pl.ds(ci * CH, CH), :] = Ech
        er = Ech.reshape(4, 4, CH, K)
        Ew_ref[:, pl.ds(ci * CH, CH), :] = jnp.concatenate(
            [er[0], er[1], er[2], er[3]], axis=-1)
        ti = lax.broadcasted_iota(jnp.int32, (B, CH, 1), 1) + ci * CH
        mk = (ti >= 1) & (ti < lens3)
        emv = jnp.where(mk, em, 0.0)
        if ci < (T // CH) // 2:
            emsum_f = emsum_f + jnp.sum(emv, axis=1)
        else:
            emsum_b = emsum_b + jnp.sum(emv, axis=1)

    Q = T // 4

    alpha0 = start_ref[...] + emis_ref[:, 0, :]
    m0 = jnp.max(alpha0, axis=1, keepdims=True)
    a = jnp.exp(alpha0 - m0)
    ca = m0 + emsum_f

    endv = end_ref[...]
    mb0 = jnp.max(endv)
    b = jnp.broadcast_to(jnp.exp(endv - mb0), (B, K))
    cb = mb0 + emsum_b

    blk = jnp.broadcast_to(jnp.exp(trans_ref[...])[None, :, None, :],
                           (4, K, 4, K))
    u_id = lax.broadcasted_iota(jnp.int32, (4, 1, 4, 1), 0)
    c_id = lax.broadcasted_iota(jnp.int32, (4, 1, 4, 1), 2)
    bd_bf = (blk * (u_id == c_id).astype(jnp.float32)).reshape(
        4 * K, 4 * K).astype(jnp.bfloat16)

    ii = lax.broadcasted_iota(jnp.int32, (4, K, 4 * K), 1)
    jj = lax.broadcasted_iota(jnp.int32, (4, K, 4 * K), 2)
    M_id = (jj % K == ii).astype(jnp.float32)
    lr = jnp.broadcast_to(lens, (B, K)).reshape(4, 4, K)
    lens_w = jnp.concatenate([lr[0], lr[1], lr[2], lr[3]], axis=-1)[:, None, :]

    def fsub(t, a):
        E = E_ref[:, t, :]
        an = jnp.dot(a.astype(jnp.bfloat16), expA_bf,
                     preferred_element_type=jnp.float32) * E
        return jnp.where(t < lens, an, a)

    def bsub(t, b):
        E = E_ref[:, t, :]
        bn = jnp.dot((b * E).astype(jnp.bfloat16), expAT_bf,
                     preferred_element_type=jnp.float32)
        return jnp.where(t < lens, bn, b)

    def msub(t, M):
        Mm = jnp.dot(M.reshape(4 * K, 4 * K).astype(jnp.bfloat16), bd_bf,
                     preferred_element_type=jnp.float32).reshape(4, K, 4 * K)
        Ew = Ew_ref[:, t, :][:, None, :]
        Mn = Mm * Ew
        return jnp.where(t < lens_w, Mn, M)

    def renorm(x, c):
        m = jnp.max(x, axis=1, keepdims=True)
        return x * (1.0 / m), c + jnp.log(m)

    def renorm_m(M, c):
        mx = jnp.max(M, axis=1, keepdims=True)
        mcs = [jnp.max(mx[:, :, K * blk_i:K * (blk_i + 1)], axis=2,
                       keepdims=True) for blk_i in range(4)]
        mxb = jnp.concatenate(
            [jnp.broadcast_to(m, (4, 1, K)) for m in mcs], axis=2)
        dlt = jnp.concatenate([m[:, :, 0] for m in mcs], axis=0)
        return M * (1.0 / mxb), c + jnp.log(dlt)

    M2 = M_id
    M3 = M_id
    cM2 = jnp.zeros((B, 1), jnp.float32)
    cM3 = jnp.zeros((B, 1), jnp.float32)
    for j in range(1, 8):
        a = fsub(j, a)
    for j in range(8):
        M2 = msub(Q + j, M2)
        M3 = msub(2 * Q + j, M3)
    for j in range(4):
        b = bsub(T - 1 - j, b)
    b, cb = renorm(b, cb)
    for j in range(4, 8):
        b = bsub(T - 1 - j, b)
    a, ca = renorm(a, ca)
    b, cb = renorm(b, cb)
    M2, cM2 = renorm_m(M2, cM2)
    M3, cM3 = renorm_m(M3, cM3)

    def group(g, carry):
        a, ca, b, cb, M2, cM2, M3, cM3 = carry
        for j in range(4):
            a = fsub(8 * g + j, a)
            b = bsub(T - 1 - 8 * g - j, b)
            M2 = msub(Q + 8 * g + j, M2)
            M3 = msub(2 * Q + 8 * g + j, M3)
        b, cb = renorm(b, cb)
        for j in range(4, 8):
            a = fsub(8 * g + j, a)
            b = bsub(T - 1 - 8 * g - j, b)
            M2 = msub(Q + 8 * g + j, M2)
            M3 = msub(2 * Q + 8 * g + j, M3)
        a, ca = renorm(a, ca)
        b, cb = renorm(b, cb)
        M2, cM2 = renorm_m(M2, cM2)
        M3, cM3 = renorm_m(M3, cM3)
        return a, ca, b, cb, M2, cM2, M3, cM3

    a, ca, b, cb, M2, cM2, M3, cM3 = lax.fori_loop(
        1, Q // 8, group, (a, ca, b, cb, M2, cM2, M3, cM3))

    def apply_m(v, M):
        rows = []
        for bt in range(B):
            u, cc = bt % 4, bt // 4
            blk_m = M[u][:, K * cc:K * (cc + 1)].astype(jnp.bfloat16)
            rows.append(jnp.dot(v[bt:bt + 1, :].astype(jnp.bfloat16), blk_m,
                                preferred_element_type=jnp.float32))
        return jnp.concatenate(rows, axis=0)

    v = apply_m(a, M2)
    v = apply_m(v, M3)
    z = (ca[:, 0] + cM2[:, 0] + cM3[:, 0] + cb[:, 0]
         + jnp.log(jnp.sum(v * b, axis=1)))
    out_ref[...] = jnp.broadcast_to(z[:, None], (B, 128))


def _tc_logz(emissions, transitions, start_transitions, end_transitions,
             len32):
    out = pl.pallas_call(
        _tc_logz_body,
        out_shape=jax.ShapeDtypeStruct((B, 128), jnp.float32),
        scratch_shapes=[pltpu.VMEM((B, T, K), jnp.float32),
                        pltpu.VMEM((4, T, 4 * K), jnp.float32)],
    )(
        emissions,
        transitions,
        transitions.T,
        start_transitions.reshape(1, K),
        end_transitions.reshape(1, K),
        len32.reshape(B, 1),
    )
    return out[:, 0]


@functools.cache
def _make_sc_scores():
    mesh = plsc.VectorSubcoreMesh(core_axis_name="c", subcore_axis_name="s")

    HT = T // 2

    @functools.partial(
        pl.kernel,
        mesh=mesh,
        compiler_params=pltpu.CompilerParams(needs_layout_passes=False),
        out_type=jax.ShapeDtypeStruct((2 * B, 16), jnp.float32),
        scratch_types=[
            pltpu.VMEM((HT, K), jnp.float32),
            pltpu.VMEM((T,), jnp.int32),
            pltpu.VMEM((K, K), jnp.float32),
            pltpu.VMEM((K,), jnp.float32),
            pltpu.VMEM((K,), jnp.float32),
            pltpu.VMEM((B,), jnp.int32),
            pltpu.VMEM((16,), jnp.float32),
            pltpu.SemaphoreType.DMA,
        ],
    )
    def _sc_scores(emis_hbm, trans_hbm, start_hbm, end_hbm, tags_hbm, len_hbm,
                   out_hbm, emis_v, tags_v, trans_v, start_v, end_v, len_v,
                   acc_v, sem):
        s_id = lax.axis_index("s")
        c_id = lax.axis_index("c")
        wid = s_id * 2 + c_id
        b = s_id
        h = c_id

        cps = [
            pltpu.async_copy(emis_hbm.at[b, pl.ds(h * HT, HT)], emis_v, sem),
            pltpu.async_copy(tags_hbm.at[b], tags_v, sem),
            pltpu.async_copy(trans_hbm, trans_v, sem),
            pltpu.async_copy(start_hbm, start_v, sem),
            pltpu.async_copy(end_hbm, end_v, sem),
            pltpu.async_copy(len_hbm, len_v, sem),
        ]
        for cp in cps:
            cp.wait()

        lane = lax.iota(jnp.int32, 16)
        bvec = jnp.full((16,), b, jnp.int32)
        lenv = plsc.load_gather(len_v, [bvec])
        hoff = h * HT

        def gath(i, acc):
            tl_ = lane + 16 * i
            tv = tl_ + hoff
            tags_t = plsc.load_gather(tags_v, [tv])
            ev = plsc.load_gather(emis_v, [tl_, tags_t])
            mk = (tv < lenv).astype(jnp.float32)
            acc = acc + ev * mk
            pv = plsc.load_gather(tags_v, [jnp.maximum(tv - 1, 0)])
            trv = plsc.load_gather(trans_v, [pv, tags_t])
            mk2 = jnp.where(tv >= 1, mk, 0.0)
            return acc + trv * mk2

        acc = lax.fori_loop(0, HT // 16, gath, jnp.zeros((16,), jnp.float32))

        lane0 = lane == 0
        zero16 = jnp.zeros((16,), jnp.int32)
        t0 = plsc.load_gather(tags_v, [zero16])
        sv = plsc.load_gather(start_v, [t0])
        smk = (lane0 & (jnp.full((16,), h) == 0)).astype(jnp.float32)
        acc = acc + sv * smk
        lastt = jnp.maximum(lenv - 1, 0)
        tlast = plsc.load_gather(tags_v, [lastt])
        evv = plsc.load_gather(end_v, [tlast])
        emk = (lane0 & (lastt // HT == jnp.full((16,), h))).astype(jnp.float32)
        acc = acc + evv * emk

        tot = jnp.sum(acc)
        acc_v[...] = jnp.full((16,), tot)
        pltpu.sync_copy(acc_v, out_hbm.at[wid])

    return _sc_scores


def kernel(emissions, transitions, start_transitions, end_transitions, tags,
           lengths):
    tags32 = tags.astype(jnp.int32)
    len32 = lengths.astype(jnp.int32)
    log_z = _tc_logz(emissions, transitions, start_transitions, end_transitions,
                     len32)
    sc_out = _make_sc_scores()(emissions, transitions, start_transitions,
                               end_transitions, tags32, len32)
    scores = sc_out[:, 0].reshape(B, 2).sum(axis=1)
    return scores - log_z
